# baseline (speedup 1.0000x reference)
import math

import jax
import jax.numpy as jnp
from jax.experimental import pallas as pl
from jax.experimental.pallas import tpu as pltpu

_MASK = -1.0
_LANES = 128


def _ceil_to(x, m):
    return ((x + m - 1) // m) * m


def _bln_pool_kernel(length, x_ref, o_ref):
    x = x_ref[...]
    tb, _, n = x.shape
    cnt = jnp.where(x != _MASK, jnp.float32(1.0), jnp.float32(0.0))
    ones = jnp.ones((tb, 8, length), jnp.float32)
    dn = (((2,), (1,)), ((0,), (0,)))
    tot = jax.lax.dot_general(ones, x, dn,
                              preferred_element_type=jnp.float32)
    c = jax.lax.dot_general(ones, cnt, dn,
                            preferred_element_type=jnp.float32)
    t0 = tot[:, 0, :]
    c0 = c[:, 0, :]
    o_ref[...] = ((t0 + (jnp.float32(length) - c0)) / c0).astype(o_ref.dtype)


def _native_pool_kernel(length, x_ref, o_ref):
    x3 = x_ref[...]
    tb, n, _ = x3.shape
    x2 = x3.reshape(tb * n, length)
    ones = jnp.ones((8, length), jnp.float32)
    cnt = jnp.where(x2 != _MASK, jnp.float32(1.0), jnp.float32(0.0))
    dn = (((1,), (1,)), ((), ()))
    tot = jax.lax.dot_general(ones, x2, dn,
                              preferred_element_type=jnp.float32)
    c = jax.lax.dot_general(ones, cnt, dn,
                            preferred_element_type=jnp.float32)
    y = (tot + (jnp.float32(length) - c)) / c
    o_ref[...] = y[0:1, :].reshape(1, 1, tb * n)


def _seg_pool_kernel(length, x_ref, seg_ref, out_ref):
    x = x_ref[...]
    seg = seg_ref[...]
    cnt = jnp.where(x != _MASK, jnp.float32(1.0), jnp.float32(0.0))
    dn = (((1,), (0,)), ((), ()))
    tot = jax.lax.dot_general(x, seg, dn,
                              preferred_element_type=jnp.float32)
    c = jax.lax.dot_general(cnt, seg, dn,
                            preferred_element_type=jnp.float32)
    s = tot + (jnp.float32(length) - c)
    out_ref[...] = (s / c).astype(out_ref.dtype)


def _rows_pool_kernel(x_ref, out_ref):
    x = x_ref[...]
    valid = x != _MASK
    c = jnp.sum(valid.astype(jnp.float32), axis=-1, keepdims=True)
    s = jnp.sum(jnp.where(valid, x, jnp.zeros_like(x)),
                axis=-1, dtype=jnp.float32, keepdims=True)
    out_ref[...] = (s / c).astype(out_ref.dtype)


def _pick_tile(rows, row_bytes, align, target_bytes=4 << 20):
    tm = max(align, (target_bytes // row_bytes) // align * align)
    if rows >= 4 * align:
        tm = min(tm, _ceil_to(pl.cdiv(rows, 4), align))
    return max(align, min(tm, _ceil_to(rows, align)))


def kernel(sims):
    B, N, L = sims.shape
    dtype = sims.dtype
    itemsize = jnp.dtype(dtype).itemsize
    M = B * N

    cparams = pltpu.CompilerParams(
        dimension_semantics=("parallel",),
        vmem_limit_bytes=48 << 20)
    cost = pl.CostEstimate(
        flops=4 * M * L, transcendentals=0,
        bytes_accessed=M * L * itemsize + M * itemsize)

    grid_t = 16
    while grid_t > 1 and (B % grid_t or (B // grid_t) % 8):
        grid_t //= 2
    if (dtype == jnp.float32 and N % _LANES == 0
            and B % grid_t == 0 and (B // grid_t) % 8 == 0):
        tb = B // grid_t
        xt = jnp.transpose(sims, (0, 2, 1))
        return pl.pallas_call(
            lambda xr, orf: _bln_pool_kernel(L, xr, orf),
            out_shape=jax.ShapeDtypeStruct((B, N), dtype),
            grid=(grid_t,),
            in_specs=[pl.BlockSpec((tb, L, N), lambda i: (i, 0, 0))],
            out_specs=pl.BlockSpec((tb, N), lambda i: (i, 0)),
            compiler_params=cparams,
            cost_estimate=cost,
        )(xt)

    grid_b = 4
    if (dtype == jnp.float32 and B % grid_b == 0
            and ((B // grid_b) * N) % _LANES == 0 and L <= 512):
        tb = B // grid_b
        lanes_out = tb * N
        out = pl.pallas_call(
            lambda xr, orf: _native_pool_kernel(L, xr, orf),
            out_shape=jax.ShapeDtypeStruct((grid_b, 1, lanes_out), dtype),
            grid=(grid_b,),
            in_specs=[pl.BlockSpec((tb, N, L), lambda i: (i, 0, 0))],
            out_specs=pl.BlockSpec((1, 1, lanes_out), lambda i: (i, 0, 0)),
            compiler_params=cparams,
            cost_estimate=cost,
        )(sims)
        return out.reshape(B, N)

    r0 = _LANES // math.gcd(L, _LANES)
    R = 0
    if M % r0 == 0:
        R = r0
        for scale in (8, 4, 2):
            cand = r0 * scale
            if cand * L <= 512 and M % cand == 0:
                R = cand
                break
    if R >= 1:
        rows, width = M // R, R * L
        x = sims.reshape(rows, width)
        tm = _pick_tile(rows, width * itemsize, 8)
        seg = (jnp.arange(width, dtype=jnp.int32)[:, None] // L
               == jnp.arange(R, dtype=jnp.int32)[None, :]).astype(jnp.float32)
        out = pl.pallas_call(
            lambda xr, sr, orf: _seg_pool_kernel(L, xr, sr, orf),
            out_shape=jax.ShapeDtypeStruct((rows, R), dtype),
            grid=(pl.cdiv(rows, tm),),
            in_specs=[pl.BlockSpec((tm, width), lambda i: (i, 0)),
                      pl.BlockSpec((width, R), lambda i: (0, 0))],
            out_specs=pl.BlockSpec((tm, R), lambda i: (i, 0)),
            compiler_params=cparams,
            cost_estimate=cost,
        )(x, seg)
        return out.reshape(B, N)

    x = sims.reshape(M, L)
    tm = _pick_tile(M, _ceil_to(L, _LANES) * itemsize, 8)
    out = pl.pallas_call(
        _rows_pool_kernel,
        out_shape=jax.ShapeDtypeStruct((M, 1), dtype),
        grid=(pl.cdiv(M, tm),),
        in_specs=[pl.BlockSpec((tm, L), lambda i: (i, 0))],
        out_specs=pl.BlockSpec((tm, 1), lambda i: (i, 0)),
        compiler_params=cparams,
        cost_estimate=cost,
    )(x)
    return out.reshape(B, N)


# speedup vs baseline: 1.4169x; 1.4169x over previous
import math

import jax
import jax.numpy as jnp
from jax.experimental import pallas as pl
from jax.experimental.pallas import tpu as pltpu

_MASK = -1.0
_LANES = 128


def _ceil_to(x, m):
    return ((x + m - 1) // m) * m


def _bln_pool_kernel(length, x_ref, o_ref):
    x = x_ref[...]
    tb, _, n = x.shape
    cnt = jnp.where(x != _MASK, jnp.float32(1.0), jnp.float32(0.0))
    ones = jnp.ones((tb, 8, length), jnp.float32)
    dn = (((2,), (1,)), ((0,), (0,)))
    tot = jax.lax.dot_general(ones, x, dn,
                              preferred_element_type=jnp.float32)
    c = jax.lax.dot_general(ones, cnt, dn,
                            preferred_element_type=jnp.float32)
    t0 = tot[:, 0, :]
    c0 = c[:, 0, :]
    o_ref[...] = ((t0 + (jnp.float32(length) - c0)) / c0).astype(o_ref.dtype)


def _native_pool_kernel(length, x_ref, o_ref):
    x3 = x_ref[...]
    tb, n, _ = x3.shape
    x2 = x3.reshape(tb * n, length)
    ones = jnp.ones((8, length), jnp.float32)
    cnt = jnp.where(x2 != _MASK, jnp.float32(1.0), jnp.float32(0.0))
    dn = (((1,), (1,)), ((), ()))
    tot = jax.lax.dot_general(ones, x2, dn,
                              preferred_element_type=jnp.float32)
    c = jax.lax.dot_general(ones, cnt, dn,
                            preferred_element_type=jnp.float32)
    y = (tot + (jnp.float32(length) - c)) / c
    o_ref[...] = y[0:1, :].reshape(1, 1, tb * n)


def _seg_pool_kernel(length, x_ref, seg_ref, out_ref):
    x = x_ref[...]
    seg = seg_ref[...]
    cnt = jnp.where(x != _MASK, jnp.float32(1.0), jnp.float32(0.0))
    dn = (((1,), (0,)), ((), ()))
    tot = jax.lax.dot_general(x, seg, dn,
                              preferred_element_type=jnp.float32)
    c = jax.lax.dot_general(cnt, seg, dn,
                            preferred_element_type=jnp.float32)
    s = tot + (jnp.float32(length) - c)
    out_ref[...] = (s / c).astype(out_ref.dtype)


def _rows_pool_kernel(x_ref, out_ref):
    x = x_ref[...]
    valid = x != _MASK
    c = jnp.sum(valid.astype(jnp.float32), axis=-1, keepdims=True)
    s = jnp.sum(jnp.where(valid, x, jnp.zeros_like(x)),
                axis=-1, dtype=jnp.float32, keepdims=True)
    out_ref[...] = (s / c).astype(out_ref.dtype)


def _pick_tile(rows, row_bytes, align, target_bytes=4 << 20):
    tm = max(align, (target_bytes // row_bytes) // align * align)
    if rows >= 4 * align:
        tm = min(tm, _ceil_to(pl.cdiv(rows, 4), align))
    return max(align, min(tm, _ceil_to(rows, align)))


def kernel(sims):
    B, N, L = sims.shape
    dtype = sims.dtype
    itemsize = jnp.dtype(dtype).itemsize
    M = B * N

    cparams = pltpu.CompilerParams(
        dimension_semantics=("parallel",),
        vmem_limit_bytes=48 << 20)
    cost = pl.CostEstimate(
        flops=4 * M * L, transcendentals=0,
        bytes_accessed=M * L * itemsize + M * itemsize)

    grid_t = 4
    while grid_t > 1 and (B % grid_t or (B // grid_t) % 8):
        grid_t //= 2
    if (dtype == jnp.float32 and N % _LANES == 0
            and B % grid_t == 0 and (B // grid_t) % 8 == 0):
        tb = B // grid_t
        xt = jnp.transpose(sims, (0, 2, 1))
        return pl.pallas_call(
            lambda xr, orf: _bln_pool_kernel(L, xr, orf),
            out_shape=jax.ShapeDtypeStruct((B, N), dtype),
            grid=(grid_t,),
            in_specs=[pl.BlockSpec((tb, L, N), lambda i: (i, 0, 0))],
            out_specs=pl.BlockSpec((tb, N), lambda i: (i, 0)),
            compiler_params=cparams,
            cost_estimate=cost,
        )(xt)

    grid_b = 4
    if (dtype == jnp.float32 and B % grid_b == 0
            and ((B // grid_b) * N) % _LANES == 0 and L <= 512):
        tb = B // grid_b
        lanes_out = tb * N
        out = pl.pallas_call(
            lambda xr, orf: _native_pool_kernel(L, xr, orf),
            out_shape=jax.ShapeDtypeStruct((grid_b, 1, lanes_out), dtype),
            grid=(grid_b,),
            in_specs=[pl.BlockSpec((tb, N, L), lambda i: (i, 0, 0))],
            out_specs=pl.BlockSpec((1, 1, lanes_out), lambda i: (i, 0, 0)),
            compiler_params=cparams,
            cost_estimate=cost,
        )(sims)
        return out.reshape(B, N)

    r0 = _LANES // math.gcd(L, _LANES)
    R = 0
    if M % r0 == 0:
        R = r0
        for scale in (8, 4, 2):
            cand = r0 * scale
            if cand * L <= 512 and M % cand == 0:
                R = cand
                break
    if R >= 1:
        rows, width = M // R, R * L
        x = sims.reshape(rows, width)
        tm = _pick_tile(rows, width * itemsize, 8)
        seg = (jnp.arange(width, dtype=jnp.int32)[:, None] // L
               == jnp.arange(R, dtype=jnp.int32)[None, :]).astype(jnp.float32)
        out = pl.pallas_call(
            lambda xr, sr, orf: _seg_pool_kernel(L, xr, sr, orf),
            out_shape=jax.ShapeDtypeStruct((rows, R), dtype),
            grid=(pl.cdiv(rows, tm),),
            in_specs=[pl.BlockSpec((tm, width), lambda i: (i, 0)),
                      pl.BlockSpec((width, R), lambda i: (0, 0))],
            out_specs=pl.BlockSpec((tm, R), lambda i: (i, 0)),
            compiler_params=cparams,
            cost_estimate=cost,
        )(x, seg)
        return out.reshape(B, N)

    x = sims.reshape(M, L)
    tm = _pick_tile(M, _ceil_to(L, _LANES) * itemsize, 8)
    out = pl.pallas_call(
        _rows_pool_kernel,
        out_shape=jax.ShapeDtypeStruct((M, 1), dtype),
        grid=(pl.cdiv(M, tm),),
        in_specs=[pl.BlockSpec((tm, L), lambda i: (i, 0))],
        out_specs=pl.BlockSpec((tm, 1), lambda i: (i, 0)),
        compiler_params=cparams,
        cost_estimate=cost,
    )(x)
    return out.reshape(B, N)
